# Initial kernel scaffold; baseline (speedup 1.0000x reference)
import functools

import jax
import jax.numpy as jnp
from jax import lax
from jax.experimental import pallas as pl
from jax.experimental.pallas import tpu as pltpu
from jax.experimental.pallas import tpu_sc as plsc

WINDOW = 61
PSIZE = 4
IMG = 64
IMG_FLAT = IMG * IMG
NQ = 961
NQ_PAD = 976
T = PSIZE * PSIZE
OUT_ROW = NQ * T
OUT_ROW_PAD = NQ_PAD * T
LANES = 16


def _sc_body(x_hbm, mask_hbm, out_hbm, x_v, out_v, mask_v, base_v):
    nc = 2
    ns = 16
    wid = lax.axis_index("s") * nc + lax.axis_index("c")
    rows_total = x_hbm.shape[0]
    rows_per_w = rows_total // (nc * ns)
    n0 = wid * rows_per_w

    iota = lax.broadcasted_iota(jnp.int32, (LANES,), 0)

    pltpu.sync_copy(mask_hbm, mask_v)

    def base_step(g, carry):
        q0 = g * LANES
        w = mask_v[pl.ds(q0, LANES)]
        rr = w // WINDOW
        cc = w - rr * WINDOW
        base_v[pl.ds(q0, LANES)] = rr * IMG + cc
        return carry

    lax.fori_loop(0, NQ_PAD // LANES, base_step, 0)

    def row_step(r, carry):
        n = n0 + r
        pltpu.sync_copy(x_hbm.at[n], x_v)

        def group_step(g, c2):
            q0 = g * LANES
            bvec = base_v[pl.ds(q0, LANES)]
            dbase = (q0 + iota) * T
            for t in range(T):
                off = IMG * (t // PSIZE) + (t % PSIZE)
                vals = plsc.load_gather(x_v, [bvec + off])
                plsc.store_scatter(out_v, [dbase + t], vals)
            return c2

        lax.fori_loop(0, NQ_PAD // LANES, group_step, 0)
        pltpu.sync_copy(out_v.at[pl.ds(0, OUT_ROW)], out_hbm.at[n])
        return carry

    lax.fori_loop(0, rows_per_w, row_step, 0)


def kernel(input_data, mask):
    B, C, H, W = input_data.shape
    rows = B * C
    x2 = input_data.reshape(rows, H * W)
    mask_p = jnp.pad(mask.astype(jnp.int32), (0, NQ_PAD - NQ))

    mesh = plsc.VectorSubcoreMesh(core_axis_name="c", subcore_axis_name="s")
    run = functools.partial(
        pl.kernel,
        mesh=mesh,
        out_type=jax.ShapeDtypeStruct((rows, OUT_ROW), jnp.float32),
        scratch_types=[
            pltpu.VMEM((IMG_FLAT,), jnp.float32),
            pltpu.VMEM((OUT_ROW_PAD,), jnp.float32),
            pltpu.VMEM((NQ_PAD,), jnp.int32),
            pltpu.VMEM((NQ_PAD,), jnp.int32),
        ],
    )(_sc_body)
    out = run(x2, mask_p)
    return out.reshape(rows, NQ, T)


# speedup vs baseline: 5.3886x; 5.3886x over previous
import functools

import jax
import jax.numpy as jnp
from jax import lax
from jax.experimental import pallas as pl
from jax.experimental.pallas import tpu as pltpu
from jax.experimental.pallas import tpu_sc as plsc

WINDOW = 61
PSIZE = 4
IMG = 64
IMG_FLAT = IMG * IMG
NQ = 961
NQ_PAD = 976
T = PSIZE * PSIZE
OUT_ROW = NQ * T
OUT_ROW_PAD = NQ_PAD * T
LANES = 16


def _sc_body(x_hbm, mask_hbm, out_hbm, x_v, out_v, mask_v, base_v):
    nc = 2
    ns = 16
    wid = lax.axis_index("s") * nc + lax.axis_index("c")
    rows_total = x_hbm.shape[0]
    rows_per_w = rows_total // (nc * ns)
    n0 = wid * rows_per_w

    iota = lax.broadcasted_iota(jnp.int32, (LANES,), 0)

    pltpu.sync_copy(mask_hbm, mask_v)

    def base_step(g, carry):
        q0 = g * LANES
        w = mask_v[pl.ds(q0, LANES)]
        rr = lax.shift_right_logical(w * 68760, 22)
        cc = w - rr * WINDOW
        base_v[pl.ds(q0, LANES)] = rr * IMG + cc
        return carry

    lax.fori_loop(0, NQ_PAD // LANES, base_step, 0)

    def row_step(r, carry):
        n = n0 + r
        pltpu.sync_copy(x_hbm.at[n], x_v)

        def group_step(g, c2):
            q0 = g * LANES
            bvec = base_v[pl.ds(q0, LANES)]
            dbase = (q0 + iota) * T
            for t in range(T):
                off = IMG * (t // PSIZE) + (t % PSIZE)
                vals = plsc.load_gather(x_v, [bvec + off])
                plsc.store_scatter(out_v, [dbase + t], vals)
            return c2

        lax.fori_loop(0, NQ_PAD // LANES, group_step, 0)
        pltpu.sync_copy(out_v.at[pl.ds(0, OUT_ROW)], out_hbm.at[n])
        return carry

    lax.fori_loop(0, rows_per_w, row_step, 0)


def kernel(input_data, mask):
    B, C, H, W = input_data.shape
    rows = B * C
    x2 = input_data.reshape(rows, H * W)
    mask_p = jnp.pad(mask.astype(jnp.int32), (0, NQ_PAD - NQ))

    mesh = plsc.VectorSubcoreMesh(core_axis_name="c", subcore_axis_name="s")
    run = functools.partial(
        pl.kernel,
        mesh=mesh,
        compiler_params=pltpu.CompilerParams(
            needs_layout_passes=False,
            use_tc_tiling_on_sc=False,
        ),
        out_type=jax.ShapeDtypeStruct((rows, OUT_ROW), jnp.float32),
        scratch_types=[
            pltpu.VMEM((IMG_FLAT,), jnp.float32),
            pltpu.VMEM((OUT_ROW_PAD,), jnp.float32),
            pltpu.VMEM((NQ_PAD,), jnp.int32),
            pltpu.VMEM((NQ_PAD,), jnp.int32),
        ],
    )(_sc_body)
    out = run(x2, mask_p)
    return out.reshape(rows, NQ, T)


# speedup vs baseline: 8.3818x; 1.5555x over previous
import functools

import jax
import jax.numpy as jnp
from jax import lax
from jax.experimental import pallas as pl
from jax.experimental.pallas import tpu as pltpu
from jax.experimental.pallas import tpu_sc as plsc

WINDOW = 61
PSIZE = 4
IMG = 64
IMG_FLAT = IMG * IMG
NQ = 961
NQ_PAD = 976
T = PSIZE * PSIZE
OUT_ROWS = NQ * T
LANES = 16
NW = 32
CQ = 4
CROWS = CQ * T
NCHUNK = 8


def _sc_body(xt_hbm, mask_hbm, out_hbm, mask_v, src_v, buf0, buf1,
             in_sem0, in_sem1, out_sem0, out_sem1):
    nc = 2
    wid = lax.axis_index("s") * nc + lax.axis_index("c")
    q_start = (NQ * wid) // NW
    q_end = (NQ * (wid + 1)) // NW

    iota = lax.broadcasted_iota(jnp.int32, (LANES,), 0)
    off_v = lax.shift_right_logical(iota, 2) * IMG + (iota & 3)

    pltpu.sync_copy(mask_hbm, mask_v)

    for c in range(NCHUNK):
        qc = jnp.minimum(q_start + CQ * c, q_end - CQ)
        for gq in range(CQ):
            qv = jnp.broadcast_to(qc + gq, (LANES,))
            w = plsc.load_gather(mask_v, [qv])
            rr = lax.shift_right_logical(w * 68760, 22)
            base = w + 3 * rr
            src_v[c, pl.ds(gq * LANES, LANES)] = base + off_v

    bufs = (buf0, buf1)
    in_sems = (in_sem0, in_sem1)
    out_sems = (out_sem0, out_sem1)

    def chunk_p0(c):
        qc = jnp.minimum(q_start + CQ * c, q_end - CQ)
        return qc * T

    copies_in = [None, None]
    copies_out = [None, None]
    copies_in[0] = pltpu.make_async_copy(
        xt_hbm.at[src_v.at[0]], bufs[0], in_sems[0])
    copies_in[0].start()
    for c in range(NCHUNK):
        b = c % 2
        nb = (c + 1) % 2
        if c + 1 < NCHUNK:
            if copies_out[nb] is not None:
                copies_out[nb].wait()
                copies_out[nb] = None
            copies_in[nb] = pltpu.make_async_copy(
                xt_hbm.at[src_v.at[c + 1]], bufs[nb], in_sems[nb])
            copies_in[nb].start()
        copies_in[b].wait()
        copies_out[b] = pltpu.make_async_copy(
            bufs[b], out_hbm.at[pl.ds(chunk_p0(c), CROWS)], out_sems[b])
        copies_out[b].start()
    for b in range(2):
        if copies_out[b] is not None:
            copies_out[b].wait()


def kernel(input_data, mask):
    B, C, H, W = input_data.shape
    rows = B * C
    xt = input_data.reshape(rows, H * W).T
    mask_p = jnp.pad(mask.astype(jnp.int32), (0, NQ_PAD - NQ))

    mesh = plsc.VectorSubcoreMesh(core_axis_name="c", subcore_axis_name="s")
    run = functools.partial(
        pl.kernel,
        mesh=mesh,
        compiler_params=pltpu.CompilerParams(
            needs_layout_passes=False,
            use_tc_tiling_on_sc=False,
        ),
        out_type=jax.ShapeDtypeStruct((OUT_ROWS, rows), jnp.float32),
        scratch_types=[
            pltpu.VMEM((NQ_PAD,), jnp.int32),
            pltpu.VMEM((NCHUNK, CROWS), jnp.int32),
            pltpu.VMEM((CROWS, rows), jnp.float32),
            pltpu.VMEM((CROWS, rows), jnp.float32),
            pltpu.SemaphoreType.DMA,
            pltpu.SemaphoreType.DMA,
            pltpu.SemaphoreType.DMA,
            pltpu.SemaphoreType.DMA,
        ],
    )(_sc_body)
    out2d = run(xt, mask_p)
    return out2d.T.reshape(rows, NQ, T)


# speedup vs baseline: 11.6694x; 1.3922x over previous
import functools

import jax
import jax.numpy as jnp
from jax import lax
from jax.experimental import pallas as pl
from jax.experimental.pallas import tpu as pltpu
from jax.experimental.pallas import tpu_sc as plsc

WINDOW = 61
PSIZE = 4
IMG = 64
NQ = 961
NQ_PAD = 976
T = PSIZE * PSIZE
OUT_ROWS = NQ * T
LANES = 16
NW = 32
CQ = 4
CROWS = CQ * T
NCHUNK = 8
LB = 4
CGROWS = CROWS * LB


def _sc_body(xt_hbm, mask_hbm, out_hbm, mask_v, src_v, src2_v, buf0, buf1,
             in_sem0, in_sem1, out_sem0, out_sem1):
    nc = 2
    wid = lax.axis_index("s") * nc + lax.axis_index("c")
    q_start = (NQ * wid) // NW
    q_end = (NQ * (wid + 1)) // NW

    iota = lax.broadcasted_iota(jnp.int32, (LANES,), 0)
    off_v = lax.shift_right_logical(iota, 2) * IMG + (iota & 3)

    pltpu.sync_copy(mask_hbm, mask_v)

    for c in range(NCHUNK):
        qc = jnp.minimum(q_start + CQ * c, q_end - CQ)
        for gq in range(CQ):
            qv = jnp.broadcast_to(qc + gq, (LANES,))
            w = plsc.load_gather(mask_v, [qv])
            rr = lax.shift_right_logical(w * 68760, 22)
            base = w + 3 * rr
            src_v[c, pl.ds(gq * LANES, LANES)] = base + off_v

    for c in range(NCHUNK):
        def build(jg, _, c=c):
            jv = jg * LANES + iota
            tp = lax.shift_right_logical(jv, 5)
            tn = lax.shift_right_logical(jv, 3) & 3
            sp = jv & 7
            p_local = tp * 8 + sp
            cv = jnp.broadcast_to(jnp.int32(c), (LANES,))
            sr = plsc.load_gather(src_v, [cv, p_local])
            h = lax.shift_right_logical(jv, 7)
            jr = jv & 127
            plsc.store_scatter(src2_v, [2 * c + h, jr], sr * LB + tn)
            return 0

        lax.fori_loop(0, CGROWS // LANES, build, 0)

    bufs = (buf0, buf1)
    in_sems = (in_sem0, in_sem1)
    out_sems = (out_sem0, out_sem1)

    def band0(c):
        qc = jnp.minimum(q_start + CQ * c, q_end - CQ)
        return qc * (T // 8) * (LB * 8)

    def start_gather(c, b):
        copies = (
            pltpu.make_async_copy(
                xt_hbm.at[src2_v.at[2 * c]],
                bufs[b].at[pl.ds(0, 128)], in_sems[b]),
            pltpu.make_async_copy(
                xt_hbm.at[src2_v.at[2 * c + 1]],
                bufs[b].at[pl.ds(128, 128)], in_sems[b]),
        )
        for cp in copies:
            cp.start()
        return copies

    copies_in = [None, None]
    copies_out = [None, None]
    copies_in[0] = start_gather(0, 0)
    for c in range(NCHUNK):
        b = c % 2
        nb = (c + 1) % 2
        if c + 1 < NCHUNK:
            if copies_out[nb] is not None:
                copies_out[nb].wait()
                copies_out[nb] = None
            copies_in[nb] = start_gather(c + 1, nb)
        for cp in copies_in[b]:
            cp.wait()
        copies_out[b] = pltpu.make_async_copy(
            bufs[b], out_hbm.at[pl.ds(band0(c), CGROWS)], out_sems[b])
        copies_out[b].start()
    for b in range(2):
        if copies_out[b] is not None:
            copies_out[b].wait()


def kernel(input_data, mask):
    B, C, H, W = input_data.shape
    rows = B * C
    xt4 = input_data.reshape(rows, H * W).T.reshape(H * W * LB, rows // LB)
    mask_p = jnp.pad(mask.astype(jnp.int32), (0, NQ_PAD - NQ))

    mesh = plsc.VectorSubcoreMesh(core_axis_name="c", subcore_axis_name="s")
    run = functools.partial(
        pl.kernel,
        mesh=mesh,
        compiler_params=pltpu.CompilerParams(
            needs_layout_passes=False,
            use_tc_tiling_on_sc=False,
        ),
        out_type=jax.ShapeDtypeStruct((OUT_ROWS * LB, rows // LB),
                                      jnp.float32),
        scratch_types=[
            pltpu.VMEM((NQ_PAD,), jnp.int32),
            pltpu.VMEM((NCHUNK, CROWS), jnp.int32),
            pltpu.VMEM((2 * NCHUNK, 128), jnp.int32),
            pltpu.VMEM((CGROWS, rows // LB), jnp.float32),
            pltpu.VMEM((CGROWS, rows // LB), jnp.float32),
            pltpu.SemaphoreType.DMA,
            pltpu.SemaphoreType.DMA,
            pltpu.SemaphoreType.DMA,
            pltpu.SemaphoreType.DMA,
        ],
    )(_sc_body)
    out4 = run(xt4, mask_p)
    out2d = (out4.reshape(OUT_ROWS // 8, LB, 8, rows // LB)
             .transpose(0, 2, 1, 3)
             .reshape(OUT_ROWS, rows))
    return out2d.T.reshape(rows, NQ, T)
